# baseline (speedup 1.0000x reference)
import functools

import jax
import jax.numpy as jnp
from jax import lax
from jax.experimental import pallas as pl
from jax.experimental.pallas import tpu as pltpu
from jax.experimental.pallas import tpu_sc as plsc

D_MODEL = 1024
D_FF = 2048
N_EXPERTS = 8
MAX_K = 4
THRESHOLD = 0.8
T_TOKENS = 2048

NW = 32
TPW = T_TOKENS // NW
XG_ROWS = N_EXPERTS * T_TOKENS
YG_ROWS = N_EXPERTS * T_TOKENS
T_BLK = 256

_MESH = dict(core_axis_name="c", subcore_axis_name="s")


def _wid():
    return lax.axis_index("s") * 2 + lax.axis_index("c")


def _lane():
    return lax.broadcasted_iota(jnp.int32, (16,), 0)


def _vscal(vec, lane, pos):
    return jnp.sum(jnp.where(lane == pos, vec, 0 if vec.dtype == jnp.int32 else 0.0))


def _gates_from_logits(lv):
    m = lv[0]
    for e in range(1, N_EXPERTS):
        m = jnp.maximum(m, lv[e])
    ex = [jnp.exp(lv[e] - m) for e in range(N_EXPERTS)]
    ssum = ex[0]
    for e in range(1, N_EXPERTS):
        ssum = ssum + ex[e]
    p = [ex[e] / ssum for e in range(N_EXPERTS)]
    gts = []
    for j in range(N_EXPERTS):
        above = jnp.zeros((16,), jnp.float32)
        csb = jnp.zeros((16,), jnp.float32)
        for e in range(N_EXPERTS):
            if e == j:
                continue
            ind = (p[e] >= p[j]) if e < j else (p[e] > p[j])
            above = above + jnp.where(ind, 1.0, 0.0)
            csb = csb + jnp.where(ind, p[e], 0.0)
        mask = (csb < THRESHOLD) & (above < float(MAX_K))
        gts.append(jnp.where(mask, p[j], 0.0))
    return gts


def _sc_route(logits_hbm, gates_hbm, cstage_hbm, lg_v, gb_v, cnt_v):
    w = _wid()
    lane = _lane()
    pltpu.sync_copy(logits_hbm.at[pl.ds(w * 512, 512)], lg_v)
    cnt_tot = [jnp.int32(0)] * N_EXPERTS
    for c2 in range(4):
        base = c2 * 128
        lv = [plsc.load_gather(lg_v, [base + lane * 8 + e])
              for e in range(N_EXPERTS)]
        gts = _gates_from_logits(lv)
        for j in range(N_EXPERTS):
            plsc.store_scatter(gb_v, [base + lane * 8 + j], gts[j])
        for e in range(N_EXPERTS):
            cnt_tot[e] = cnt_tot[e] + jnp.sum((gts[e] > 0.0).astype(jnp.int32))
    cvec = jnp.zeros((16,), jnp.int32)
    for e in range(N_EXPERTS):
        cvec = jnp.where(lane == e, cnt_tot[e], cvec)
    cnt_v[...] = cvec
    pltpu.sync_copy(gb_v, gates_hbm.at[pl.ds(w * 512, 512)])
    pltpu.sync_copy(cnt_v, cstage_hbm.at[pl.ds(w * 16, 16)])


def _sc_gather(gates_hbm, cstage_hbm, xb_hbm, xg_hbm, counts_hbm, ridx_hbm,
               gate4_hbm, gb_v, call_v, lidx_v, xb0_v, xb1_v, ridx_l, g4_l,
               cnt_v, sem0, sem1):
    w = _wid()
    lane = _lane()
    base_t = w * TPW
    pltpu.sync_copy(gates_hbm.at[pl.ds(w * 512, 512)], gb_v)
    pltpu.sync_copy(cstage_hbm, call_v)

    gbase = []
    tot = []
    for e in range(N_EXPERTS):
        v0 = plsc.load_gather(call_v, [lane * 16 + e])
        v1 = plsc.load_gather(call_v, [(lane + 16) * 16 + e])
        cs0 = plsc.cumsum(v0)
        cs1 = plsc.cumsum(v1)
        tot0 = jnp.sum(v0)
        tot_e = tot0 + jnp.sum(v1)
        a = _vscal(cs0, lane, w)
        b = _vscal(cs1, lane, w - 16)
        incl = jnp.where(w < 16, a, tot0 + b)
        myc_sp = plsc.load_gather(call_v, [lane * 0 + (w * 16 + e)])
        myc = _vscal(myc_sp, lane, 0)
        gbase.append(incl - myc)
        tot.append(tot_e)

    zi = jnp.zeros((16,), jnp.int32)
    for i in range(16):
        ridx_l[pl.ds(i * 16, 16)] = zi
        g4_l[pl.ds(i * 16, 16)] = jnp.zeros((16,), jnp.float32)
    for i in range(NW):
        lidx_v[pl.ds(i * 16, 16)] = zi

    kc = [zi] * 4
    cnts = []
    for e in range(N_EXPERTS):
        off_e = jnp.int32(0)
        for c2 in range(4):
            gv = plsc.load_gather(gb_v, [c2 * 128 + lane * 8 + e])
            mk = gv > 0.0
            mi = mk.astype(jnp.int32)
            pos = off_e + plsc.cumsum(mi) - 1
            tid = base_t + c2 * 16 + lane
            plsc.store_scatter(lidx_v, [e * TPW + pos], tid, mask=mk)
            rv = e * T_TOKENS + gbase[e] + pos
            kidx = (c2 * 16 + lane) * MAX_K + kc[c2]
            plsc.store_scatter(ridx_l, [kidx], rv, mask=mk)
            plsc.store_scatter(g4_l, [kidx], gv, mask=mk)
            kc[c2] = kc[c2] + mi
            off_e = off_e + jnp.sum(mi)
        cnts.append(off_e)

    bufs = [xb0_v, xb1_v]
    sems = [sem0, sem1]

    def gstart(e):
        return pltpu.async_copy(
            xb_hbm.at[lidx_v.at[pl.ds(e * TPW, TPW)]], bufs[e % 2],
            sems[e % 2])

    cp = gstart(0)
    for e in range(N_EXPERTS):
        cp.wait()
        if e < N_EXPERTS - 1:
            cp = gstart(e + 1)
        off = jnp.int32(0)
        dst0 = e * T_TOKENS + gbase[e]
        for k in (6, 5, 4, 3, 2, 1, 0):
            sz = 1 << k
            bit = (cnts[e] >> k) & 1

            @pl.when(bit == 1)
            def _(off=off, sz=sz, e=e, dst0=dst0):
                pltpu.sync_copy(bufs[e % 2].at[pl.ds(off, sz), :],
                                xg_hbm.at[pl.ds(dst0 + off, sz), :])

            off = off + bit * sz

    pltpu.sync_copy(ridx_l, ridx_hbm.at[pl.ds(w * TPW * MAX_K, TPW * MAX_K)])
    pltpu.sync_copy(g4_l, gate4_hbm.at[pl.ds(w * TPW * MAX_K, TPW * MAX_K)])

    @pl.when(w == 0)
    def _():
        tv = jnp.zeros((16,), jnp.int32)
        for e in range(N_EXPERTS):
            tv = jnp.where(lane == e, tot[e], tv)
        cnt_v[...] = tv
        pltpu.sync_copy(cnt_v, counts_hbm)


def _ffn_kernel(cnt_ref, xg_ref, wg_ref, wu_ref, wd_ref, yg_ref):
    e = pl.program_id(0)
    tb = pl.program_id(1)
    count = cnt_ref[e]

    @pl.when(tb * T_BLK < count)
    def _():
        xb = xg_ref[...]
        hg = jnp.dot(xb, wg_ref[0], preferred_element_type=jnp.float32)
        hu = jnp.dot(xb, wu_ref[0], preferred_element_type=jnp.float32)
        h = (hg * jax.nn.sigmoid(hg) * hu).astype(jnp.bfloat16)
        yg_ref[...] = jnp.dot(h, wd_ref[0], preferred_element_type=jnp.float32)


def _sc_combine(yg_hbm, ridx_hbm, gate4_hbm, out_hbm, rv_v, gv4_v, ybuf_v,
                obuf_v, sem):
    w = _wid()
    lane = _lane()
    pltpu.sync_copy(ridx_hbm.at[pl.ds(w * TPW * MAX_K, TPW * MAX_K)], rv_v)
    pltpu.sync_copy(gate4_hbm.at[pl.ds(w * TPW * MAX_K, TPW * MAX_K)], gv4_v)

    def group(g, carry):
        iv = rv_v[pl.ds(g * 16, 16)]
        pltpu.async_copy(yg_hbm.at[iv], ybuf_v, sem).wait()
        gsp = []
        for ti in range(4):
            for j in range(MAX_K):
                gs = plsc.load_gather(gv4_v, [lane * 0 + (g * 16 + ti * 4 + j)])
                gsp.append(gs)

        def chunk(ch, c2):
            sl = pl.ds(ch * 16, 16)
            for ti in range(4):
                acc = gsp[ti * 4] * ybuf_v[ti * 4, sl]
                for j in range(1, MAX_K):
                    acc = acc + gsp[ti * 4 + j] * ybuf_v[ti * 4 + j, sl]
                obuf_v[ti, sl] = acc
            return c2

        lax.fori_loop(0, D_MODEL // 16, chunk, 0)
        pltpu.sync_copy(obuf_v, out_hbm.at[pl.ds(w * TPW + g * 4, 4), :])
        return carry

    lax.fori_loop(0, TPW // 4, group, 0)


@jax.jit
def kernel(x, W_router, W_gate, W_up, W_down):
    logits = (x @ W_router).reshape(-1)
    xb = x.astype(jnp.bfloat16)
    wg = W_gate.astype(jnp.bfloat16)
    wu = W_up.astype(jnp.bfloat16)
    wd = W_down.astype(jnp.bfloat16)

    mesh = plsc.VectorSubcoreMesh(**_MESH)

    sc_params = pltpu.CompilerParams(use_tc_tiling_on_sc=False, needs_layout_passes=False)
    route = pl.kernel(
        _sc_route,
        mesh=mesh,
        compiler_params=sc_params,
        out_type=(
            jax.ShapeDtypeStruct((T_TOKENS * N_EXPERTS,), jnp.float32),
            jax.ShapeDtypeStruct((NW * 16,), jnp.int32),
        ),
        scratch_types=[
            pltpu.VMEM((512,), jnp.float32),
            pltpu.VMEM((512,), jnp.float32),
            pltpu.VMEM((16,), jnp.int32),
        ],
    )
    gates_flat, cstage = route(logits)

    gather = pl.kernel(
        _sc_gather,
        mesh=mesh,
        compiler_params=sc_params,
        out_type=(
            jax.ShapeDtypeStruct((XG_ROWS, D_MODEL // 2), jnp.int32),
            jax.ShapeDtypeStruct((16,), jnp.int32),
            jax.ShapeDtypeStruct((T_TOKENS * MAX_K,), jnp.int32),
            jax.ShapeDtypeStruct((T_TOKENS * MAX_K,), jnp.float32),
        ),
        scratch_types=[
            pltpu.VMEM((512,), jnp.float32),
            pltpu.VMEM((512,), jnp.int32),
            pltpu.VMEM((NW * 16,), jnp.int32),
            pltpu.VMEM((TPW, D_MODEL // 2), jnp.int32),
            pltpu.VMEM((TPW, D_MODEL // 2), jnp.int32),
            pltpu.VMEM((TPW * MAX_K,), jnp.int32),
            pltpu.VMEM((TPW * MAX_K,), jnp.float32),
            pltpu.VMEM((16,), jnp.int32),
            pltpu.SemaphoreType.DMA,
            pltpu.SemaphoreType.DMA,
        ],
    )
    xb_i = lax.bitcast_convert_type(
        xb.reshape(T_TOKENS, D_MODEL // 2, 2), jnp.int32)
    xg_i, counts, ridx, gate4 = gather(gates_flat, cstage, xb_i)
    xg = lax.bitcast_convert_type(xg_i, jnp.bfloat16).reshape(
        XG_ROWS, D_MODEL)

    nb = T_TOKENS // T_BLK
    yg = pl.pallas_call(
        _ffn_kernel,
        grid_spec=pltpu.PrefetchScalarGridSpec(
            num_scalar_prefetch=1,
            grid=(N_EXPERTS, T_TOKENS // T_BLK),
            in_specs=[
                pl.BlockSpec((T_BLK, D_MODEL), lambda e, tb, s: (e * nb + tb, 0)),
                pl.BlockSpec((1, D_MODEL, D_FF), lambda e, tb, s: (e, 0, 0)),
                pl.BlockSpec((1, D_MODEL, D_FF), lambda e, tb, s: (e, 0, 0)),
                pl.BlockSpec((1, D_FF, D_MODEL), lambda e, tb, s: (e, 0, 0)),
            ],
            out_specs=pl.BlockSpec(
                (T_BLK, D_MODEL),
                lambda e, tb, s: (e * (T_TOKENS // T_BLK) + tb, 0)),
        ),
        out_shape=jax.ShapeDtypeStruct((YG_ROWS, D_MODEL), jnp.float32),
        compiler_params=pltpu.CompilerParams(
            dimension_semantics=("arbitrary", "arbitrary"),
        ),
    )(counts, xg, wg, wu, wd)

    combine = pl.kernel(
        _sc_combine,
        mesh=mesh,
        compiler_params=sc_params,
        out_type=jax.ShapeDtypeStruct((T_TOKENS, D_MODEL), jnp.float32),
        scratch_types=[
            pltpu.VMEM((TPW * MAX_K,), jnp.int32),
            pltpu.VMEM((TPW * MAX_K,), jnp.float32),
            pltpu.VMEM((16, D_MODEL), jnp.float32),
            pltpu.VMEM((4, D_MODEL), jnp.float32),
            pltpu.SemaphoreType.DMA,
        ],
    )
    out = combine(yg, ridx, gate4)
    return out


# speedup vs baseline: 2.8915x; 2.8915x over previous
import functools

import jax
import jax.numpy as jnp
from jax import lax
from jax.experimental import pallas as pl
from jax.experimental.pallas import tpu as pltpu
from jax.experimental.pallas import tpu_sc as plsc

D_MODEL = 1024
D_FF = 2048
N_EXPERTS = 8
MAX_K = 4
THRESHOLD = 0.8
T_TOKENS = 2048

NW = 32
TPW = T_TOKENS // NW
F_BLK = 1024


def _lane():
    return lax.broadcasted_iota(jnp.int32, (16,), 0)


def _gates_from_logits(lv):
    m = lv[0]
    for e in range(1, N_EXPERTS):
        m = jnp.maximum(m, lv[e])
    ex = [jnp.exp(lv[e] - m) for e in range(N_EXPERTS)]
    ssum = ex[0]
    for e in range(1, N_EXPERTS):
        ssum = ssum + ex[e]
    p = [ex[e] / ssum for e in range(N_EXPERTS)]
    gts = []
    for j in range(N_EXPERTS):
        above = jnp.zeros((16,), jnp.float32)
        csb = jnp.zeros((16,), jnp.float32)
        for e in range(N_EXPERTS):
            if e == j:
                continue
            ind = (p[e] >= p[j]) if e < j else (p[e] > p[j])
            above = above + jnp.where(ind, 1.0, 0.0)
            csb = csb + jnp.where(ind, p[e], 0.0)
        mask = (csb < THRESHOLD) & (above < float(MAX_K))
        gts.append(jnp.where(mask, p[j], 0.0))
    return gts


def _sc_route(logits_hbm, gates_hbm, lg_v, gb_v):
    w = lax.axis_index("s") * 2 + lax.axis_index("c")
    lane = _lane()
    pltpu.sync_copy(logits_hbm.at[pl.ds(w * 512, 512)], lg_v)
    for c2 in range(4):
        base = c2 * 128
        lv = [plsc.load_gather(lg_v, [base + lane * 8 + e])
              for e in range(N_EXPERTS)]
        gts = _gates_from_logits(lv)
        for j in range(N_EXPERTS):
            plsc.store_scatter(gb_v, [base + lane * 8 + j], gts[j])
    pltpu.sync_copy(gb_v, gates_hbm.at[pl.ds(w * 512, 512)])


def _ffn_kernel(x_ref, g_ref, wg_ref, wu_ref, wd_ref, out_ref):
    e = pl.program_id(0)
    f = pl.program_id(1)

    x = x_ref[...]
    hg = jnp.dot(x, wg_ref[0], preferred_element_type=jnp.float32)
    hu = jnp.dot(x, wu_ref[0], preferred_element_type=jnp.float32)
    h = (hg * jax.nn.sigmoid(hg) * hu).astype(jnp.bfloat16)
    y = jnp.dot(h, wd_ref[0], preferred_element_type=jnp.float32)

    gates = g_ref[...]
    lane = jax.lax.broadcasted_iota(jnp.int32, gates.shape, 1)
    g = jnp.sum(jnp.where(lane == e, gates, 0.0), axis=-1, keepdims=True)
    contrib = y * g

    @pl.when((e == 0) & (f == 0))
    def _():
        out_ref[...] = contrib

    @pl.when((e != 0) | (f != 0))
    def _():
        out_ref[...] = out_ref[...] + contrib


@jax.jit
def kernel(x, W_router, W_gate, W_up, W_down):
    logits = (x @ W_router).reshape(-1)
    xb = x.astype(jnp.bfloat16)
    wg = W_gate.astype(jnp.bfloat16)
    wu = W_up.astype(jnp.bfloat16)
    wd = W_down.astype(jnp.bfloat16)

    route = pl.kernel(
        _sc_route,
        mesh=plsc.VectorSubcoreMesh(core_axis_name="c", subcore_axis_name="s"),
        out_type=jax.ShapeDtypeStruct((T_TOKENS * N_EXPERTS,), jnp.float32),
        scratch_types=[
            pltpu.VMEM((512,), jnp.float32),
            pltpu.VMEM((512,), jnp.float32),
        ],
        compiler_params=pltpu.CompilerParams(
            use_tc_tiling_on_sc=False, needs_layout_passes=False),
    )
    gates = route(logits).reshape(T_TOKENS, N_EXPERTS)

    out = pl.pallas_call(
        _ffn_kernel,
        grid=(N_EXPERTS, D_FF // F_BLK),
        in_specs=[
            pl.BlockSpec((T_TOKENS, D_MODEL), lambda e, f: (0, 0)),
            pl.BlockSpec((T_TOKENS, N_EXPERTS), lambda e, f: (0, 0)),
            pl.BlockSpec((1, D_MODEL, F_BLK), lambda e, f: (e, 0, f)),
            pl.BlockSpec((1, D_MODEL, F_BLK), lambda e, f: (e, 0, f)),
            pl.BlockSpec((1, F_BLK, D_MODEL), lambda e, f: (e, f, 0)),
        ],
        out_specs=pl.BlockSpec((T_TOKENS, D_MODEL), lambda e, f: (0, 0)),
        out_shape=jax.ShapeDtypeStruct((T_TOKENS, D_MODEL), jnp.float32),
        compiler_params=pltpu.CompilerParams(
            dimension_semantics=("arbitrary", "arbitrary"),
        ),
    )(xb, gates, wg, wu, wd)
    return out
